# Initial kernel scaffold; baseline (speedup 1.0000x reference)
import jax
import jax.numpy as jnp
from jax.experimental import pallas as pl


def kernel(seq1, seq2, adj, sparse, msk, samp_bias1, samp_bias2, W_gcn, b_gcn, prelu_a, W_disc, b_disc):
    raise NotImplementedError("write your pallas kernel here")


# speedup vs baseline: 6.8852x; 6.8852x over previous
import functools

import jax
import jax.numpy as jnp
from jax import lax
from jax.experimental import pallas as pl
from jax.experimental.pallas import tpu as pltpu
from jax.experimental.pallas import tpu_sc as plsc

_NUM_TILES = 16
_CHUNK = 80


def _matmul_kernel(x_ref, w_ref, b_ref, oa_ref, ob_ref):
    h = (
        jnp.dot(x_ref[...], w_ref[...], preferred_element_type=jnp.float32)
        + b_ref[...]
    )
    nh = h.shape[1]
    oa_ref[...] = h[:, : nh // 2]
    ob_ref[...] = h[:, nh // 2:]


def _post_kernel(a1a_ref, a1b_ref, a2a_ref, a2b_ref, dg_ref, m_ref,
                 sb1_ref, sb2_ref, wd_ref, bd_ref, pa_ref, o1_ref, o2_ref):
    deg = jnp.maximum(dg_ref[:, 0:1], 1.0)
    pa = pa_ref[0, 0]
    a1 = jnp.concatenate([a1a_ref[...], a1b_ref[...]], axis=1) / deg
    h1 = jnp.where(a1 > 0, a1, pa * a1)
    a2 = jnp.concatenate([a2a_ref[...], a2b_ref[...]], axis=1) / deg
    h2 = jnp.where(a2 > 0, a2, pa * a2)
    m = m_ref[...]
    c = jnp.sum(m * h1, axis=0, keepdims=True) / jnp.sum(m)
    c = jax.nn.sigmoid(c)
    cw = jnp.sum(wd_ref[...] * c, axis=1, keepdims=True)
    bias = bd_ref[0, 0]
    o1_ref[...] = (
        jnp.dot(h1, cw, preferred_element_type=jnp.float32) + bias + sb1_ref[...]
    )
    o2_ref[...] = (
        jnp.dot(h2, cw, preferred_element_type=jnp.float32) + bias + sb2_ref[...]
    )


def _sc_aggregate(tab_a, tab_b, src3, srcn3, dst3, z_h, z_16, ones_h,
                  n_pad, cpt, rpt):
    hw = tab_a.shape[1]

    def body(ta_ref, tb_ref, src_ref, srcn_ref, dst_ref, zh_ref, z16_ref,
             ones_ref, agg_a_ref, agg_b_ref, deg_ref,
             src_v, srcn_v, dst_v, rows_v, ones_v, acc_sh, deg_sh, sem):
        c = lax.axis_index("c")
        s = lax.axis_index("s")
        sl = pl.ds(s * rpt, rpt)
        pltpu.sync_copy(src_ref.at[s], src_v)
        pltpu.sync_copy(srcn_ref.at[s], srcn_v)
        pltpu.sync_copy(dst_ref.at[s], dst_v)
        pltpu.sync_copy(ones_ref, ones_v)
        pltpu.sync_copy(zh_ref, acc_sh.at[sl])
        pltpu.sync_copy(z16_ref, deg_sh.at[sl])
        plsc.subcore_barrier()

        def make_chunk(table_ref, idx_v, with_deg):
            def chunk(j, carry):
                pltpu.async_copy(table_ref.at[idx_v.at[j]], rows_v, sem).wait()
                pltpu.sync_copy(rows_v, acc_sh.at[dst_v.at[j]], add=True)
                if with_deg:
                    pltpu.sync_copy(ones_v, deg_sh.at[dst_v.at[j]], add=True)
                return carry
            return chunk

        def phase(table_ref, idx_v, out_base, with_deg, write_deg):
            lax.fori_loop(0, cpt, make_chunk(table_ref, idx_v, with_deg), 0)
            plsc.subcore_barrier()
            osl = pl.ds(out_base + s * rpt, rpt)
            if write_deg:
                @pl.when(c == 0)
                def _():
                    pltpu.sync_copy(deg_sh.at[sl], deg_ref.at[sl])
            @pl.when(c == 0)
            def _():
                pltpu.sync_copy(acc_sh.at[sl], agg_a_ref.at[osl])
            @pl.when(c == 1)
            def _():
                pltpu.sync_copy(acc_sh.at[sl], agg_b_ref.at[osl])
            pltpu.sync_copy(zh_ref, acc_sh.at[sl])
            plsc.subcore_barrier()

        @pl.when(c == 0)
        def _():
            phase(ta_ref, src_v, 0, True, True)
            phase(ta_ref, srcn_v, n_pad, False, False)

        @pl.when(c == 1)
        def _():
            phase(tb_ref, src_v, 0, True, False)
            phase(tb_ref, srcn_v, n_pad, False, False)

    mesh = plsc.VectorSubcoreMesh(core_axis_name="c", subcore_axis_name="s")
    run = pl.kernel(
        body,
        compiler_params=pltpu.CompilerParams(use_tc_tiling_on_sc=False),
        out_type=[
            jax.ShapeDtypeStruct((2 * n_pad, hw), jnp.float32),
            jax.ShapeDtypeStruct((2 * n_pad, hw), jnp.float32),
            jax.ShapeDtypeStruct((n_pad, 16), jnp.float32),
        ],
        mesh=mesh,
        scratch_types=[
            pltpu.VMEM((cpt, _CHUNK), jnp.int32),
            pltpu.VMEM((cpt, _CHUNK), jnp.int32),
            pltpu.VMEM((cpt, _CHUNK), jnp.int32),
            pltpu.VMEM((_CHUNK, hw), jnp.float32),
            pltpu.VMEM((_CHUNK, 16), jnp.float32),
            pltpu.VMEM_SHARED((n_pad, hw), jnp.float32),
            pltpu.VMEM_SHARED((n_pad, 16), jnp.float32),
            pltpu.SemaphoreType.DMA,
        ],
    )
    return run(tab_a, tab_b, src3, srcn3, dst3, z_h, z_16, ones_h)


def kernel(seq1, seq2, adj, sparse, msk, samp_bias1, samp_bias2,
           W_gcn, b_gcn, prelu_a, W_disc, b_disc):
    n = seq1.shape[1]
    d = seq1.shape[2]
    nh = W_gcn.shape[1] // 2
    hw = nh // 2
    e = adj.shape[1]
    assert e % (_NUM_TILES * _CHUNK) == 0
    cpt = e // (_NUM_TILES * _CHUNK)
    rpt = -(-n // (_NUM_TILES * 8)) * 8
    n_pad = rpt * _NUM_TILES

    w_e = W_gcn[:, 0::2]
    b_e = b_gcn[0::2][None, :]

    seqs = jnp.concatenate([seq1[0], seq2[0]], axis=0)
    bm = 2000
    tab_a, tab_b = pl.pallas_call(
        _matmul_kernel,
        grid=((2 * n) // bm,),
        in_specs=[
            pl.BlockSpec((bm, d), lambda i: (i, 0)),
            pl.BlockSpec((d, nh), lambda i: (0, 0)),
            pl.BlockSpec((1, nh), lambda i: (0, 0)),
        ],
        out_specs=[
            pl.BlockSpec((bm, hw), lambda i: (i, 0)),
            pl.BlockSpec((bm, hw), lambda i: (i, 0)),
        ],
        out_shape=[
            jax.ShapeDtypeStruct((2 * n, hw), jnp.float32),
            jax.ShapeDtypeStruct((2 * n, hw), jnp.float32),
        ],
    )(seqs, w_e, b_e)

    src3 = adj[0].reshape(_NUM_TILES, cpt, _CHUNK)
    srcn3 = src3 + n
    dst3 = adj[1].reshape(_NUM_TILES, cpt, _CHUNK)
    z_h = jnp.zeros((rpt, hw), jnp.float32)
    z_16 = jnp.zeros((rpt, 16), jnp.float32)
    ones_h = jnp.ones((_CHUNK, 16), jnp.float32)
    agg_a, agg_b, deg = _sc_aggregate(tab_a, tab_b, src3, srcn3, dst3,
                                      z_h, z_16, ones_h, n_pad, cpt, rpt)

    o1, o2 = pl.pallas_call(
        _post_kernel,
        compiler_params=pltpu.CompilerParams(vmem_limit_bytes=100 * 2**20),
        out_shape=[
            jax.ShapeDtypeStruct((n, 1), jnp.float32),
            jax.ShapeDtypeStruct((n, 1), jnp.float32),
        ],
    )(agg_a[:n], agg_b[:n], agg_a[n_pad:n_pad + n], agg_b[n_pad:n_pad + n],
      deg[:n], msk[0][:, None], samp_bias1[0][:, None],
      samp_bias2[0][:, None], W_disc, b_disc.reshape(1, 1),
      prelu_a.reshape(1, 1))
    return jnp.concatenate([o1[:, 0], o2[:, 0]])[None, :]


# speedup vs baseline: 10.8626x; 1.5777x over previous
import functools

import jax
import jax.numpy as jnp
from jax import lax
from jax.experimental import pallas as pl
from jax.experimental.pallas import tpu as pltpu
from jax.experimental.pallas import tpu_sc as plsc

_NUM_TILES = 16
_CHUNK = 80


def _matmul_kernel(x_ref, w_ref, b_ref, oa_ref, ob_ref):
    h = (
        jnp.dot(x_ref[...], w_ref[...], preferred_element_type=jnp.float32)
        + b_ref[...]
    )
    nh = h.shape[1]
    oa_ref[...] = h[:, : nh // 2]
    ob_ref[...] = h[:, nh // 2:]


def _post_kernel(a1a_ref, a1b_ref, a2a_ref, a2b_ref, dg_ref, m_ref,
                 sb1_ref, sb2_ref, wd_ref, bd_ref, pa_ref, o1_ref, o2_ref):
    deg = jnp.maximum(dg_ref[:, 0:1], 1.0)
    pa = pa_ref[0, 0]
    a1 = jnp.concatenate([a1a_ref[...], a1b_ref[...]], axis=1) / deg
    h1 = jnp.where(a1 > 0, a1, pa * a1)
    a2 = jnp.concatenate([a2a_ref[...], a2b_ref[...]], axis=1) / deg
    h2 = jnp.where(a2 > 0, a2, pa * a2)
    m = m_ref[...]
    c = jnp.sum(m * h1, axis=0, keepdims=True) / jnp.sum(m)
    c = jax.nn.sigmoid(c)
    cw = jnp.sum(wd_ref[...] * c, axis=1, keepdims=True)
    bias = bd_ref[0, 0]
    o1_ref[...] = (
        jnp.dot(h1, cw, preferred_element_type=jnp.float32) + bias + sb1_ref[...]
    )
    o2_ref[...] = (
        jnp.dot(h2, cw, preferred_element_type=jnp.float32) + bias + sb2_ref[...]
    )


def _sc_aggregate(tab_a, tab_b, src3, srcn3, dst3, z_h, z_16, ones_h,
                  n_pad, cpt, rpt):
    hw = tab_a.shape[1]

    def body(ta_ref, tb_ref, src_ref, srcn_ref, dst_ref, zh_ref, z16_ref,
             ones_ref, agg_a_ref, agg_b_ref, deg_ref,
             src_v, srcn_v, dst_v, rows0_v, rows1_v, ones_v, acc_sh, deg_sh,
             sem0, sem1):
        c = lax.axis_index("c")
        s = lax.axis_index("s")
        sl = pl.ds(s * rpt, rpt)
        pltpu.sync_copy(src_ref.at[s], src_v)
        pltpu.sync_copy(srcn_ref.at[s], srcn_v)
        pltpu.sync_copy(dst_ref.at[s], dst_v)
        pltpu.sync_copy(ones_ref, ones_v)
        pltpu.sync_copy(zh_ref, acc_sh.at[sl])
        pltpu.sync_copy(z16_ref, deg_sh.at[sl])
        plsc.subcore_barrier()

        def run_chunks(table_ref, idx_v, with_deg):
            pltpu.async_copy(table_ref.at[idx_v.at[0]], rows0_v, sem0)

            def consume(e, buf, bsem):
                pltpu.make_async_copy(table_ref.at[idx_v.at[e]], buf, bsem).wait()
                pltpu.sync_copy(buf, acc_sh.at[dst_v.at[e]], add=True)
                if with_deg:
                    pltpu.sync_copy(ones_v, deg_sh.at[dst_v.at[e]], add=True)

            def pair(j2, carry):
                e0 = 2 * j2
                e1 = e0 + 1
                pltpu.async_copy(table_ref.at[idx_v.at[e1]], rows1_v, sem1)
                consume(e0, rows0_v, sem0)

                @pl.when(e1 + 1 < cpt)
                def _():
                    pltpu.async_copy(table_ref.at[idx_v.at[e1 + 1]], rows0_v,
                                     sem0)
                consume(e1, rows1_v, sem1)
                return carry

            lax.fori_loop(0, cpt // 2, pair, 0)

        def phase(table_ref, idx_v, out_base, with_deg, write_deg):
            run_chunks(table_ref, idx_v, with_deg)
            plsc.subcore_barrier()
            osl = pl.ds(out_base + s * rpt, rpt)
            if write_deg:
                @pl.when(c == 0)
                def _():
                    pltpu.sync_copy(deg_sh.at[sl], deg_ref.at[sl])
            @pl.when(c == 0)
            def _():
                pltpu.sync_copy(acc_sh.at[sl], agg_a_ref.at[osl])
            @pl.when(c == 1)
            def _():
                pltpu.sync_copy(acc_sh.at[sl], agg_b_ref.at[osl])
            pltpu.sync_copy(zh_ref, acc_sh.at[sl])
            plsc.subcore_barrier()

        @pl.when(c == 0)
        def _():
            phase(ta_ref, src_v, 0, True, True)
            phase(ta_ref, srcn_v, n_pad, False, False)

        @pl.when(c == 1)
        def _():
            phase(tb_ref, src_v, 0, True, False)
            phase(tb_ref, srcn_v, n_pad, False, False)

    mesh = plsc.VectorSubcoreMesh(core_axis_name="c", subcore_axis_name="s")
    run = pl.kernel(
        body,
        compiler_params=pltpu.CompilerParams(use_tc_tiling_on_sc=False),
        out_type=[
            jax.ShapeDtypeStruct((2 * n_pad, hw), jnp.float32),
            jax.ShapeDtypeStruct((2 * n_pad, hw), jnp.float32),
            jax.ShapeDtypeStruct((n_pad, 16), jnp.float32),
        ],
        mesh=mesh,
        scratch_types=[
            pltpu.VMEM((cpt, _CHUNK), jnp.int32),
            pltpu.VMEM((cpt, _CHUNK), jnp.int32),
            pltpu.VMEM((cpt, _CHUNK), jnp.int32),
            pltpu.VMEM((_CHUNK, hw), jnp.float32),
            pltpu.VMEM((_CHUNK, hw), jnp.float32),
            pltpu.VMEM((_CHUNK, 16), jnp.float32),
            pltpu.VMEM_SHARED((n_pad, hw), jnp.float32),
            pltpu.VMEM_SHARED((n_pad, 16), jnp.float32),
            pltpu.SemaphoreType.DMA,
            pltpu.SemaphoreType.DMA,
        ],
    )
    return run(tab_a, tab_b, src3, srcn3, dst3, z_h, z_16, ones_h)


def kernel(seq1, seq2, adj, sparse, msk, samp_bias1, samp_bias2,
           W_gcn, b_gcn, prelu_a, W_disc, b_disc):
    n = seq1.shape[1]
    d = seq1.shape[2]
    nh = W_gcn.shape[1] // 2
    hw = nh // 2
    e = adj.shape[1]
    assert e % (_NUM_TILES * _CHUNK) == 0
    cpt = e // (_NUM_TILES * _CHUNK)
    rpt = -(-n // (_NUM_TILES * 8)) * 8
    n_pad = rpt * _NUM_TILES

    w_e = W_gcn[:, 0::2]
    b_e = b_gcn[0::2][None, :]

    seqs = jnp.concatenate([seq1[0], seq2[0]], axis=0)
    bm = 2000
    tab_a, tab_b = pl.pallas_call(
        _matmul_kernel,
        grid=((2 * n) // bm,),
        in_specs=[
            pl.BlockSpec((bm, d), lambda i: (i, 0)),
            pl.BlockSpec((d, nh), lambda i: (0, 0)),
            pl.BlockSpec((1, nh), lambda i: (0, 0)),
        ],
        out_specs=[
            pl.BlockSpec((bm, hw), lambda i: (i, 0)),
            pl.BlockSpec((bm, hw), lambda i: (i, 0)),
        ],
        out_shape=[
            jax.ShapeDtypeStruct((2 * n, hw), jnp.float32),
            jax.ShapeDtypeStruct((2 * n, hw), jnp.float32),
        ],
    )(seqs, w_e, b_e)

    src3 = adj[0].reshape(_NUM_TILES, cpt, _CHUNK)
    srcn3 = src3 + n
    dst3 = adj[1].reshape(_NUM_TILES, cpt, _CHUNK)
    z_h = jnp.zeros((rpt, hw), jnp.float32)
    z_16 = jnp.zeros((rpt, 16), jnp.float32)
    ones_h = jnp.ones((_CHUNK, 16), jnp.float32)
    agg_a, agg_b, deg = _sc_aggregate(tab_a, tab_b, src3, srcn3, dst3,
                                      z_h, z_16, ones_h, n_pad, cpt, rpt)

    o1, o2 = pl.pallas_call(
        _post_kernel,
        compiler_params=pltpu.CompilerParams(vmem_limit_bytes=100 * 2**20),
        out_shape=[
            jax.ShapeDtypeStruct((n, 1), jnp.float32),
            jax.ShapeDtypeStruct((n, 1), jnp.float32),
        ],
    )(agg_a[:n], agg_b[:n], agg_a[n_pad:n_pad + n], agg_b[n_pad:n_pad + n],
      deg[:n], msk[0][:, None], samp_bias1[0][:, None],
      samp_bias2[0][:, None], W_disc, b_disc.reshape(1, 1),
      prelu_a.reshape(1, 1))
    return jnp.concatenate([o1[:, 0], o2[:, 0]])[None, :]


# speedup vs baseline: 13.0461x; 1.2010x over previous
import functools

import jax
import jax.numpy as jnp
from jax import lax
from jax.experimental import pallas as pl
from jax.experimental.pallas import tpu as pltpu
from jax.experimental.pallas import tpu_sc as plsc

_NUM_TILES = 16
_CHUNK = 80


def _matmul_kernel(x_ref, w_ref, b_ref, oa_ref, ob_ref):
    h = (
        jnp.dot(x_ref[...], w_ref[...], preferred_element_type=jnp.float32)
        + b_ref[...]
    )
    nh = h.shape[1]
    oa_ref[...] = h[:, : nh // 2]
    ob_ref[...] = h[:, nh // 2:]


def _prelu_h(aa, ab, deg, pa):
    a = jnp.concatenate([aa, ab], axis=1) / deg
    return jnp.where(a > 0, a, pa * a)


def _readout_kernel(a1a_ref, a1b_ref, dga_ref, dgb_ref, m_ref, pa_ref,
                    csum_ref, msum_ref):
    @pl.when(pl.program_id(0) == 0)
    def _():
        csum_ref[...] = jnp.zeros_like(csum_ref)
        msum_ref[...] = jnp.zeros_like(msum_ref)

    deg = jnp.maximum(dga_ref[...] + dgb_ref[...], 1.0)
    h1 = _prelu_h(a1a_ref[...], a1b_ref[...], deg, pa_ref[0, 0])
    m = m_ref[...]
    csum_ref[...] += jnp.sum(m * h1, axis=0, keepdims=True)
    msum_ref[...] += jnp.sum(m, axis=0, keepdims=True)


def _cw_kernel(csum_ref, msum_ref, wd_ref, cw_ref):
    c = jax.nn.sigmoid(csum_ref[...] / msum_ref[0, 0])
    cw_ref[...] = jnp.sum(wd_ref[...] * c, axis=1, keepdims=True)


def _score_kernel(a1a_ref, a1b_ref, a2a_ref, a2b_ref, dga_ref, dgb_ref,
                  sb1_ref, sb2_ref, cw_ref, bd_ref, pa_ref, o1_ref, o2_ref):
    deg = jnp.maximum(dga_ref[...] + dgb_ref[...], 1.0)
    pa = pa_ref[0, 0]
    cw = cw_ref[...]
    bias = bd_ref[0, 0]
    h1 = _prelu_h(a1a_ref[...], a1b_ref[...], deg, pa)
    o1_ref[...] = (
        jnp.dot(h1, cw, preferred_element_type=jnp.float32) + bias + sb1_ref[...]
    )
    h2 = _prelu_h(a2a_ref[...], a2b_ref[...], deg, pa)
    o2_ref[...] = (
        jnp.dot(h2, cw, preferred_element_type=jnp.float32) + bias + sb2_ref[...]
    )


def _sc_aggregate(tab_a, tab_b, src3, srcn3, dst3, z_h, ones_h,
                  n_acc, cpt, rpt):
    hw = tab_a.shape[1]

    nbuf = 5
    nt = _NUM_TILES
    last = n_acc - (nt - 1) * rpt
    assert cpt % nbuf == 0 and (cpt // 2) % nbuf == 0
    assert 0 < last <= rpt and last % 8 == 0

    def body(ta_ref, tb_ref, src_ref, srcn_ref, dst_ref, zh_ref,
             ones_ref, agg_a_ref, agg_b_ref,
             src_v, srcn_v, dst_v, rows_bufs, ones_v, acc_sh, sems):
        c = lax.axis_index("c")
        s = lax.axis_index("s")

        def each_stripe(do):
            @pl.when(s < nt - 1)
            def _():
                do(s * rpt, rpt)

            @pl.when(s == nt - 1)
            def _():
                do((nt - 1) * rpt, last)

        pltpu.sync_copy(src_ref.at[s], src_v)
        pltpu.sync_copy(srcn_ref.at[s], srcn_v)
        pltpu.sync_copy(dst_ref.at[s], dst_v)
        pltpu.sync_copy(ones_ref, ones_v)
        each_stripe(lambda off, ln: pltpu.sync_copy(
            zh_ref.at[pl.ds(0, ln)], acc_sh.at[pl.ds(off, ln)]))
        plsc.subcore_barrier()

        def run_chunks(table_ref, idx_v):
            for b in range(nbuf):
                pltpu.async_copy(table_ref.at[idx_v.at[b]], rows_bufs[b],
                                 sems[b])

            def gat_start(e, b):
                pltpu.async_copy(table_ref.at[idx_v.at[e]], rows_bufs[b],
                                 sems[b])

            def gat_wait(e, b):
                pltpu.make_async_copy(table_ref.at[idx_v.at[e]], rows_bufs[b],
                                      sems[b]).wait()

            def scat_start(e, b):
                pltpu.async_copy(rows_bufs[b], acc_sh.at[dst_v.at[e]],
                                 sems[b], add=True)

            def scat_wait(e, b):
                pltpu.make_async_copy(rows_bufs[b], acc_sh.at[dst_v.at[e]],
                                      sems[b]).wait()

            def block(k, carry):
                for b in range(nbuf):
                    e = nbuf * k + b
                    gat_wait(e, b)
                    scat_start(e, b)
                    bp = (b - 1) % nbuf
                    ep = e - 1

                    @pl.when(ep >= 0)
                    def _():
                        scat_wait(ep, bp)

                        @pl.when(ep + nbuf < cpt)
                        def _():
                            gat_start(ep + nbuf, bp)
                return carry

            lax.fori_loop(0, cpt // nbuf, block, 0)
            scat_wait(cpt - 1, (cpt - 1) % nbuf)

        def writeout(out_base, rezero):
            plsc.subcore_barrier()

            def drain(off, ln):
                asl = pl.ds(off, ln)
                osl = pl.ds(out_base + off, ln)

                @pl.when(c == 0)
                def _():
                    pltpu.sync_copy(acc_sh.at[asl], agg_a_ref.at[osl])

                @pl.when(c == 1)
                def _():
                    pltpu.sync_copy(acc_sh.at[asl], agg_b_ref.at[osl])
                if rezero:
                    pltpu.sync_copy(zh_ref.at[pl.ds(0, ln)], acc_sh.at[asl])

            each_stripe(drain)
            if rezero:
                plsc.subcore_barrier()

        def phase(table_ref, idx_v, out_base):
            run_chunks(table_ref, idx_v)
            writeout(out_base, True)

        def deg_phase():
            half = cpt // 2
            off = c * half

            def dscat_start(e, b):
                pltpu.async_copy(ones_v, acc_sh.at[dst_v.at[e]], sems[b],
                                 add=True)

            def dscat_wait(e, b):
                pltpu.make_async_copy(ones_v, acc_sh.at[dst_v.at[e]],
                                      sems[b]).wait()

            def dblock(k, carry):
                for b in range(nbuf):
                    e = off + nbuf * k + b

                    @pl.when(k > 0)
                    def _():
                        dscat_wait(e - nbuf, b)
                    dscat_start(e, b)
                return carry

            lax.fori_loop(0, half // nbuf, dblock, 0)
            for b in range(nbuf):
                dscat_wait(off + half - nbuf + b, b)
            writeout(2 * n_acc, False)

        @pl.when(c == 0)
        def _():
            phase(ta_ref, src_v, 0)
            phase(ta_ref, srcn_v, n_acc)

        @pl.when(c == 1)
        def _():
            phase(tb_ref, src_v, 0)
            phase(tb_ref, srcn_v, n_acc)

        deg_phase()

    mesh = plsc.VectorSubcoreMesh(core_axis_name="c", subcore_axis_name="s")
    run = pl.kernel(
        body,
        compiler_params=pltpu.CompilerParams(use_tc_tiling_on_sc=False),
        out_type=[
            jax.ShapeDtypeStruct((3 * n_acc, hw), jnp.float32),
            jax.ShapeDtypeStruct((3 * n_acc, hw), jnp.float32),
        ],
        mesh=mesh,
        scratch_types=[
            pltpu.VMEM((cpt, _CHUNK), jnp.int32),
            pltpu.VMEM((cpt, _CHUNK), jnp.int32),
            pltpu.VMEM((cpt, _CHUNK), jnp.int32),
            [pltpu.VMEM((_CHUNK, hw), jnp.float32) for _ in range(nbuf)],
            pltpu.VMEM((_CHUNK, hw), jnp.float32),
            pltpu.VMEM_SHARED((n_acc, hw), jnp.float32),
            [pltpu.SemaphoreType.DMA for _ in range(nbuf)],
        ],
    )
    return run(tab_a, tab_b, src3, srcn3, dst3, z_h, ones_h)


def kernel(seq1, seq2, adj, sparse, msk, samp_bias1, samp_bias2,
           W_gcn, b_gcn, prelu_a, W_disc, b_disc):
    n = seq1.shape[1]
    d = seq1.shape[2]
    nh = W_gcn.shape[1] // 2
    hw = nh // 2
    e = adj.shape[1]
    assert e % (_NUM_TILES * _CHUNK) == 0
    cpt = e // (_NUM_TILES * _CHUNK)
    rpt = -(-n // (_NUM_TILES * 8)) * 8
    n_acc = -(-n // 8) * 8

    w_e = W_gcn[:, 0::2]
    b_e = b_gcn[0::2][None, :]

    seqs = jnp.concatenate([seq1[0], seq2[0]], axis=0)
    bm = 2000
    tab_a, tab_b = pl.pallas_call(
        _matmul_kernel,
        grid=((2 * n) // bm,),
        in_specs=[
            pl.BlockSpec((bm, d), lambda i: (i, 0)),
            pl.BlockSpec((d, nh), lambda i: (0, 0)),
            pl.BlockSpec((1, nh), lambda i: (0, 0)),
        ],
        out_specs=[
            pl.BlockSpec((bm, hw), lambda i: (i, 0)),
            pl.BlockSpec((bm, hw), lambda i: (i, 0)),
        ],
        out_shape=[
            jax.ShapeDtypeStruct((2 * n, hw), jnp.float32),
            jax.ShapeDtypeStruct((2 * n, hw), jnp.float32),
        ],
    )(seqs, w_e, b_e)

    src3 = adj[0].reshape(_NUM_TILES, cpt, _CHUNK)
    srcn3 = src3 + n
    dst3 = adj[1].reshape(_NUM_TILES, cpt, _CHUNK)
    z_h = jnp.zeros((rpt, hw), jnp.float32)
    ones_h = jnp.ones((_CHUNK, hw), jnp.float32)
    agg_a, agg_b = _sc_aggregate(tab_a, tab_b, src3, srcn3, dst3,
                                 z_h, ones_h, n_acc, cpt, rpt)

    a1a, a1b = agg_a[:n], agg_b[:n]
    a2a, a2b = agg_a[n_acc:n_acc + n], agg_b[n_acc:n_acc + n]
    dga = agg_a[2 * n_acc:2 * n_acc + n, 0:1]
    dgb = agg_b[2 * n_acc:2 * n_acc + n, 0:1]
    m_col = msk[0][:, None]
    pa = prelu_a.reshape(1, 1)
    bn = 2000
    grid = (n // bn,)
    row_spec = pl.BlockSpec((bn, hw), lambda i: (i, 0))
    col_spec = pl.BlockSpec((bn, 1), lambda i: (i, 0))
    fix = lambda shape: pl.BlockSpec(shape, lambda i: (0, 0))

    csum, msum = pl.pallas_call(
        _readout_kernel,
        grid=grid,
        in_specs=[row_spec, row_spec, col_spec, col_spec, col_spec,
                  fix((1, 1))],
        out_specs=[fix((1, nh)), fix((1, 1))],
        out_shape=[
            jax.ShapeDtypeStruct((1, nh), jnp.float32),
            jax.ShapeDtypeStruct((1, 1), jnp.float32),
        ],
    )(a1a, a1b, dga, dgb, m_col, pa)

    cw = pl.pallas_call(
        _cw_kernel,
        out_shape=jax.ShapeDtypeStruct((nh, 1), jnp.float32),
    )(csum, msum, W_disc)

    o1, o2 = pl.pallas_call(
        _score_kernel,
        grid=grid,
        in_specs=[row_spec, row_spec, row_spec, row_spec, col_spec, col_spec,
                  col_spec, col_spec, fix((nh, 1)), fix((1, 1)), fix((1, 1))],
        out_specs=[col_spec, col_spec],
        out_shape=[
            jax.ShapeDtypeStruct((n, 1), jnp.float32),
            jax.ShapeDtypeStruct((n, 1), jnp.float32),
        ],
    )(a1a, a1b, a2a, a2b, dga, dgb, samp_bias1[0][:, None],
      samp_bias2[0][:, None], cw, b_disc.reshape(1, 1), pa)
    return jnp.concatenate([o1[:, 0], o2[:, 0]])[None, :]
